# Initial kernel scaffold; baseline (speedup 1.0000x reference)
import jax
import jax.numpy as jnp
from jax.experimental import pallas as pl


def kernel(data, merge, structure, W_lin, b_lin, W_tr, b_tr):
    raise NotImplementedError("write your pallas kernel here")


# speedup vs baseline: 13.0197x; 13.0197x over previous
import functools

import jax
import jax.numpy as jnp
from jax import lax
from jax.experimental import pallas as pl
from jax.experimental.pallas import tpu as pltpu
from jax.experimental.pallas import tpu_sc as plsc

NC = 2
NS = 16
NW = NC * NS
CHUNK = 128
PACK = 14


def _sc_segment_sum(data, packed, n_pad, nch):
    d = data.shape[1]
    stripe = n_pad // NS
    mesh = plsc.VectorSubcoreMesh(core_axis_name="c", subcore_axis_name="s")

    @functools.partial(
        pl.kernel,
        mesh=mesh,
        compiler_params=pltpu.CompilerParams(needs_layout_passes=False),
        out_type=[
            jax.ShapeDtypeStruct((NC, n_pad, d), jnp.float32),
            jax.ShapeDtypeStruct((NW, n_pad), jnp.float32),
        ],
        scratch_types=[
            pltpu.VMEM((nch * CHUNK,), jnp.int32),
            pltpu.VMEM((CHUNK,), jnp.int32),
            pltpu.VMEM((CHUNK,), jnp.int32),
            pltpu.VMEM((CHUNK, d), jnp.float32),
            pltpu.VMEM((n_pad,), jnp.float32),
            pltpu.VMEM_SHARED((n_pad, d), jnp.float32),
            pltpu.SemaphoreType.DMA,
        ],
    )
    def k(data_h, pk_h, zacc_h, acc_o, cnt_o,
          pk_v, sidx_v, didx_v, rows_v, hist_v, acc_s, sem):
        c = lax.axis_index("c")
        s = lax.axis_index("s")
        wid = c * NS + s
        row0 = s * stripe

        pltpu.sync_copy(zacc_h.at[pl.ds(row0, stripe)],
                        acc_s.at[pl.ds(row0, stripe)])
        zeros16 = jnp.zeros((16,), jnp.float32)

        def zhist(i, carry):
            hist_v[pl.ds(i * 16, 16)] = zeros16
            return carry

        lax.fori_loop(0, n_pad // 16, zhist, 0)
        pltpu.sync_copy(pk_h.at[wid], pk_v)
        plsc.subcore_barrier()

        mask = jnp.int32((1 << PACK) - 1)
        ones16 = jnp.ones((16,), jnp.float32)

        def chunk(i, carry):
            base = i * CHUNK
            for j in range(CHUNK // 16):
                p = pk_v[pl.ds(base + j * 16, 16)]
                dstv = lax.bitwise_and(p, mask)
                sidx_v[pl.ds(j * 16, 16)] = lax.shift_right_logical(p, PACK)
                didx_v[pl.ds(j * 16, 16)] = dstv
                plsc.addupdate_scatter(hist_v, [dstv], ones16)
            pltpu.async_copy(data_h.at[sidx_v], rows_v, sem).wait()
            pltpu.sync_copy(rows_v, acc_s.at[didx_v], add=True)
            return carry

        lax.fori_loop(0, nch, chunk, 0)
        plsc.subcore_barrier()

        pltpu.sync_copy(acc_s.at[pl.ds(row0, stripe)],
                        acc_o.at[c, pl.ds(row0, stripe)])
        pltpu.sync_copy(hist_v, cnt_o.at[wid])

    zacc = jnp.zeros((n_pad, d), jnp.float32)
    return k(data, packed, zacc)


def _tc_combine(data, merge, acc, cnt, w_lin, w_tr, b_tr):
    n, d = data.shape
    blk = 2048
    grid = -(-n // blk)
    bt2 = b_tr.reshape(1, d)
    onesw = jnp.ones((NW, 1), jnp.float32)

    def body(d_r, m_r, a_r, c_r, wl_r, wt_r, bt_r, ow_r, o_r):
        a = a_r[0, :, :] + a_r[1, :, :]
        deg = lax.dot_general(c_r[...], ow_r[...], (((0,), (0,)), ((), ())),
                              preferred_element_type=jnp.float32)
        factor = 1.0 / jnp.maximum(deg, 1.0)
        ind = (deg > 0.0).astype(jnp.float32)
        z = d_r[...] * ind - a * factor
        r = lax.dot_general(z, wl_r[...], (((1,), (1,)), ((), ())),
                            preferred_element_type=jnp.float32)
        r = r + lax.dot_general(m_r[...], wt_r[...], (((1,), (1,)), ((), ())),
                                preferred_element_type=jnp.float32)
        o_r[...] = jnp.maximum(r + bt_r[...], 0.0)

    return pl.pallas_call(
        body,
        grid=(grid,),
        in_specs=[
            pl.BlockSpec((blk, d), lambda i: (i, 0)),
            pl.BlockSpec((blk, d), lambda i: (i, 0)),
            pl.BlockSpec((NC, blk, d), lambda i: (0, i, 0)),
            pl.BlockSpec((NW, blk), lambda i: (0, i)),
            pl.BlockSpec((d, d), lambda i: (0, 0)),
            pl.BlockSpec((d, d), lambda i: (0, 0)),
            pl.BlockSpec((1, d), lambda i: (0, 0)),
            pl.BlockSpec((NW, 1), lambda i: (0, 0)),
        ],
        out_specs=pl.BlockSpec((blk, d), lambda i: (i, 0)),
        out_shape=jax.ShapeDtypeStruct((n, d), jnp.float32),
    )(data, merge, acc, cnt, w_lin, w_tr, bt2, onesw)


def kernel(data, merge, structure, W_lin, b_lin, W_tr, b_tr):
    n, d = data.shape
    e = structure.shape[1]

    nch = -(-e // (NW * CHUNK))
    e_pad = NW * nch * CHUNK
    n_pad = NS * 8 * (-(-(n + 1) // (NS * 8)))

    src = structure[0]
    dst = structure[1]
    pad = e_pad - e
    packed = jnp.concatenate(
        [lax.shift_left(src, PACK) | dst,
         jnp.full((pad,), n, jnp.int32)]).reshape(NW, nch * CHUNK)

    acc, cnt = _sc_segment_sum(data, packed, n_pad, nch)
    return _tc_combine(data, merge, acc, cnt, W_lin, W_tr, b_tr)


# speedup vs baseline: 13.8538x; 1.0641x over previous
import functools

import jax
import jax.numpy as jnp
from jax import lax
from jax.experimental import pallas as pl
from jax.experimental.pallas import tpu as pltpu
from jax.experimental.pallas import tpu_sc as plsc

NC = 2
NS = 16
NW = NC * NS
CHUNK = 64
PACK = 14


def _sc_segment_sum(data, packed, n_pad, nch):
    d = data.shape[1]
    stripe = n_pad // NS
    mesh = plsc.VectorSubcoreMesh(core_axis_name="c", subcore_axis_name="s")

    @functools.partial(
        pl.kernel,
        mesh=mesh,
        compiler_params=pltpu.CompilerParams(needs_layout_passes=False),
        out_type=[
            jax.ShapeDtypeStruct((NC, n_pad, d), jnp.float32),
            jax.ShapeDtypeStruct((NW, n_pad), jnp.float32),
        ],
        scratch_types=[
            pltpu.VMEM(((nch + 1) * CHUNK,), jnp.int32),
            pltpu.VMEM((2, CHUNK), jnp.int32),
            pltpu.VMEM((2, CHUNK), jnp.int32),
            pltpu.VMEM((2, CHUNK, d), jnp.float32),
            pltpu.VMEM((n_pad,), jnp.float32),
            pltpu.VMEM_SHARED((n_pad, d), jnp.float32),
            pltpu.SemaphoreType.DMA,
        ],
    )
    def k(data_h, pk_h, acc_o, cnt_o,
          pk_v, sidx_v, didx_v, rows_v, hist_v, acc_s, sem):
        c = lax.axis_index("c")
        s = lax.axis_index("s")
        wid = c * NS + s
        row0 = s * stripe

        zeros16 = jnp.zeros((16,), jnp.float32)

        def zhist(i, carry):
            hist_v[pl.ds(i * 16, 16)] = zeros16
            return carry

        lax.fori_loop(0, n_pad // 16, zhist, 0)

        def zrows(i, carry):
            for j in range(d // 16):
                rows_v[0, i, pl.ds(j * 16, 16)] = zeros16
            return carry

        lax.fori_loop(0, CHUNK, zrows, 0)

        nblk = n_pad // CHUNK

        def zacc(r, carry):
            bi = s + r * NS

            @pl.when(bi < nblk)
            def _():
                pltpu.sync_copy(rows_v.at[0],
                                acc_s.at[pl.ds(bi * CHUNK, CHUNK)])

            return carry

        lax.fori_loop(0, -(-nblk // NS), zacc, 0)
        pltpu.sync_copy(pk_h.at[wid], pk_v.at[pl.ds(0, nch * CHUNK)])
        dummy = jnp.full((16,), n_pad - 8, jnp.int32)
        for j in range(CHUNK // 16):
            pk_v[pl.ds(nch * CHUNK + j * 16, 16)] = dummy
        plsc.subcore_barrier()

        mask = jnp.int32((1 << PACK) - 1)
        ones16 = jnp.ones((16,), jnp.float32)

        def unpack(ci, sl):
            base = ci * CHUNK
            for j in range(CHUNK // 16):
                p = pk_v[pl.ds(base + j * 16, 16)]
                dstv = lax.bitwise_and(p, mask)
                sidx_v[sl, pl.ds(j * 16, 16)] = lax.shift_right_logical(p, PACK)
                didx_v[sl, pl.ds(j * 16, 16)] = dstv
                plsc.addupdate_scatter(hist_v, [dstv], ones16)

        def pair(kk, carry):
            c0 = 2 * kk
            unpack(c0, 0)
            d0 = pltpu.async_copy(data_h.at[sidx_v.at[0]], rows_v.at[0], sem)
            unpack(c0 + 1, 1)
            d1 = pltpu.async_copy(data_h.at[sidx_v.at[1]], rows_v.at[1], sem)
            d0.wait()
            pltpu.sync_copy(rows_v.at[0], acc_s.at[didx_v.at[0]], add=True)
            d1.wait()
            pltpu.sync_copy(rows_v.at[1], acc_s.at[didx_v.at[1]], add=True)
            return carry

        lax.fori_loop(0, nch // 2, pair, 0)
        plsc.subcore_barrier()

        pltpu.sync_copy(acc_s.at[pl.ds(row0, stripe)],
                        acc_o.at[c, pl.ds(row0, stripe)])
        pltpu.sync_copy(hist_v, cnt_o.at[wid])

    return k(data, packed)


def _tc_combine(data, merge, acc, cnt, w_lin, w_tr, b_tr):
    n, d = data.shape
    blk = 2048
    grid = -(-n // blk)
    bt2 = b_tr.reshape(1, d)
    onesw = jnp.ones((NW, 1), jnp.float32)

    def body(d_r, m_r, a_r, c_r, wl_r, wt_r, bt_r, ow_r, o_r):
        a = a_r[0, :, :] + a_r[1, :, :]
        deg = lax.dot_general(c_r[...], ow_r[...], (((0,), (0,)), ((), ())),
                              preferred_element_type=jnp.float32)
        factor = 1.0 / jnp.maximum(deg, 1.0)
        ind = (deg > 0.0).astype(jnp.float32)
        z = d_r[...] * ind - a * factor
        r = lax.dot_general(z, wl_r[...], (((1,), (1,)), ((), ())),
                            preferred_element_type=jnp.float32)
        r = r + lax.dot_general(m_r[...], wt_r[...], (((1,), (1,)), ((), ())),
                                preferred_element_type=jnp.float32)
        o_r[...] = jnp.maximum(r + bt_r[...], 0.0)

    return pl.pallas_call(
        body,
        grid=(grid,),
        in_specs=[
            pl.BlockSpec((blk, d), lambda i: (i, 0)),
            pl.BlockSpec((blk, d), lambda i: (i, 0)),
            pl.BlockSpec((NC, blk, d), lambda i: (0, i, 0)),
            pl.BlockSpec((NW, blk), lambda i: (0, i)),
            pl.BlockSpec((d, d), lambda i: (0, 0)),
            pl.BlockSpec((d, d), lambda i: (0, 0)),
            pl.BlockSpec((1, d), lambda i: (0, 0)),
            pl.BlockSpec((NW, 1), lambda i: (0, 0)),
        ],
        out_specs=pl.BlockSpec((blk, d), lambda i: (i, 0)),
        out_shape=jax.ShapeDtypeStruct((n, d), jnp.float32),
    )(data, merge, acc, cnt, w_lin, w_tr, bt2, onesw)


def kernel(data, merge, structure, W_lin, b_lin, W_tr, b_tr):
    n, d = data.shape
    e = structure.shape[1]

    nch = -(-e // (NW * CHUNK))
    nch = nch + (nch % 2)
    e_pad = NW * nch * CHUNK
    n_pad = NS * 8 * (-(-(n + 1) // (NS * 8)))

    src = structure[0]
    dst = structure[1]
    pad = e_pad - e
    packed = jnp.concatenate(
        [lax.shift_left(src, PACK) | dst,
         jnp.full((pad,), n, jnp.int32)]).reshape(NW, nch * CHUNK)

    acc, cnt = _sc_segment_sum(data, packed, n_pad, nch)
    return _tc_combine(data, merge, acc, cnt, W_lin, W_tr, b_tr)


# speedup vs baseline: 13.8946x; 1.0029x over previous
import functools

import jax
import jax.numpy as jnp
from jax import lax
from jax.experimental import pallas as pl
from jax.experimental.pallas import tpu as pltpu
from jax.experimental.pallas import tpu_sc as plsc

NC = 2
NS = 16
NW = NC * NS
CHUNK = 64
PACK = 14


def _sc_segment_sum(data, packed, n_pad, nch):
    d = data.shape[1]
    stripe = n_pad // NS
    mesh = plsc.VectorSubcoreMesh(core_axis_name="c", subcore_axis_name="s")

    @functools.partial(
        pl.kernel,
        mesh=mesh,
        compiler_params=pltpu.CompilerParams(needs_layout_passes=False),
        out_type=[
            jax.ShapeDtypeStruct((NC, n_pad, d), jnp.float32),
            jax.ShapeDtypeStruct((NW, n_pad), jnp.float32),
        ],
        scratch_types=[
            pltpu.VMEM(((nch + 1) * CHUNK,), jnp.int32),
            pltpu.VMEM((2, CHUNK), jnp.int32),
            pltpu.VMEM((2, CHUNK), jnp.int32),
            pltpu.VMEM((2, CHUNK, d), jnp.float32),
            pltpu.VMEM((n_pad,), jnp.float32),
            pltpu.VMEM_SHARED((n_pad, d), jnp.float32),
            pltpu.SemaphoreType.DMA,
        ],
    )
    def k(data_h, pk_h, acc_o, cnt_o,
          pk_v, sidx_v, didx_v, rows_v, hist_v, acc_s, sem):
        c = lax.axis_index("c")
        s = lax.axis_index("s")
        wid = c * NS + s
        row0 = s * stripe

        zeros16 = jnp.zeros((16,), jnp.float32)

        def zhist(i, carry):
            hist_v[pl.ds(i * 16, 16)] = zeros16
            return carry

        lax.fori_loop(0, n_pad // 16, zhist, 0)

        def zrows(i, carry):
            for j in range(d // 16):
                rows_v[0, i, pl.ds(j * 16, 16)] = zeros16
            return carry

        lax.fori_loop(0, CHUNK, zrows, 0)

        nblk = n_pad // CHUNK

        def zacc(r, carry):
            bi = s + r * NS

            @pl.when(bi < nblk)
            def _():
                pltpu.sync_copy(rows_v.at[0],
                                acc_s.at[pl.ds(bi * CHUNK, CHUNK)])

            return carry

        lax.fori_loop(0, -(-nblk // NS), zacc, 0)
        pltpu.sync_copy(pk_h.at[wid], pk_v.at[pl.ds(0, nch * CHUNK)])
        dummy = jnp.full((16,), n_pad - 8, jnp.int32)
        for j in range(CHUNK // 16):
            pk_v[pl.ds(nch * CHUNK + j * 16, 16)] = dummy
        plsc.subcore_barrier()

        mask = jnp.int32((1 << PACK) - 1)
        ones16 = jnp.ones((16,), jnp.float32)

        def unpack(ci, sl):
            base = ci * CHUNK
            for j in range(CHUNK // 16):
                p = pk_v[pl.ds(base + j * 16, 16)]
                dstv = lax.bitwise_and(p, mask)
                sidx_v[sl, pl.ds(j * 16, 16)] = lax.shift_right_logical(p, PACK)
                didx_v[sl, pl.ds(j * 16, 16)] = dstv
                plsc.addupdate_scatter(hist_v, [dstv], ones16)

        def pair(kk, carry):
            c0 = 2 * kk
            unpack(c0, 0)
            d0 = pltpu.async_copy(data_h.at[sidx_v.at[0]], rows_v.at[0], sem)
            unpack(c0 + 1, 1)
            d1 = pltpu.async_copy(data_h.at[sidx_v.at[1]], rows_v.at[1], sem)
            d0.wait()
            pltpu.sync_copy(rows_v.at[0], acc_s.at[didx_v.at[0]], add=True)
            d1.wait()
            pltpu.sync_copy(rows_v.at[1], acc_s.at[didx_v.at[1]], add=True)
            return carry

        lax.fori_loop(0, nch // 2, pair, 0)
        plsc.subcore_barrier()

        pltpu.sync_copy(acc_s.at[pl.ds(row0, stripe)],
                        acc_o.at[c, pl.ds(row0, stripe)])
        pltpu.sync_copy(hist_v, cnt_o.at[wid])

    return k(data, packed)


def _tc_combine(data, merge, acc, cnt, w_lin, w_tr, b_tr):
    n, d = data.shape
    blk = 2048
    grid = -(-n // blk)
    bt2 = b_tr.reshape(1, d)
    onesw = jnp.ones((NW, 1), jnp.float32)

    def body(d_r, m_r, a_r, c_r, wl_r, wt_r, bt_r, ow_r, o_r):
        a = a_r[0, :, :] + a_r[1, :, :]
        deg = lax.dot_general(c_r[...], ow_r[...], (((0,), (0,)), ((), ())),
                              preferred_element_type=jnp.float32)
        factor = 1.0 / jnp.maximum(deg, 1.0)
        ind = (deg > 0.0).astype(jnp.float32)
        z = d_r[...] * ind - a * factor
        r = lax.dot_general(z, wl_r[...], (((1,), (1,)), ((), ())),
                            preferred_element_type=jnp.float32)
        r = r + lax.dot_general(m_r[...], wt_r[...], (((1,), (1,)), ((), ())),
                                preferred_element_type=jnp.float32)
        o_r[...] = jnp.maximum(r + bt_r[...], 0.0)

    return pl.pallas_call(
        body,
        grid=(grid,),
        in_specs=[
            pl.BlockSpec((blk, d), lambda i: (i, 0)),
            pl.BlockSpec((blk, d), lambda i: (i, 0)),
            pl.BlockSpec((NC, blk, d), lambda i: (0, i, 0)),
            pl.BlockSpec((NW, blk), lambda i: (0, i)),
            pl.BlockSpec((d, d), lambda i: (0, 0)),
            pl.BlockSpec((d, d), lambda i: (0, 0)),
            pl.BlockSpec((1, d), lambda i: (0, 0)),
            pl.BlockSpec((NW, 1), lambda i: (0, 0)),
        ],
        out_specs=pl.BlockSpec((blk, d), lambda i: (i, 0)),
        out_shape=jax.ShapeDtypeStruct((n, d), jnp.float32),
    )(data, merge, acc, cnt, w_lin, w_tr, bt2, onesw)


def kernel(data, merge, structure, W_lin, b_lin, W_tr, b_tr):
    n, d = data.shape
    e = structure.shape[1]

    nch = -(-e // (NW * CHUNK))
    nch = nch + (nch % 2)
    e_pad = NW * nch * CHUNK
    n_pad = NS * 8 * (-(-(n + 1) // (NS * 8)))

    src = structure[0]
    dst = structure[1]
    pad = e_pad - e
    pad_dst = n + jnp.arange(pad, dtype=jnp.int32) % (n_pad - n)
    packed = jnp.concatenate(
        [lax.shift_left(src, PACK) | dst, pad_dst]).reshape(NW, nch * CHUNK)

    acc, cnt = _sc_segment_sum(data, packed, n_pad, nch)
    return _tc_combine(data, merge, acc, cnt, W_lin, W_tr, b_tr)


# speedup vs baseline: 18.7798x; 1.3516x over previous
import functools

import jax
import jax.numpy as jnp
from jax import lax
from jax.experimental import pallas as pl
from jax.experimental.pallas import tpu as pltpu
from jax.experimental.pallas import tpu_sc as plsc

NC = 2
NS = 16
NW = NC * NS
CHUNK = 64
PACK = 14


def _sc_segment_sum(data, packed, n_pad, nch0, nch1):
    nch = nch0
    d = data.shape[1]
    stripe = n_pad // NS
    mesh = plsc.VectorSubcoreMesh(core_axis_name="c", subcore_axis_name="s")

    @functools.partial(
        pl.kernel,
        mesh=mesh,
        compiler_params=pltpu.CompilerParams(needs_layout_passes=False),
        out_type=[
            jax.ShapeDtypeStruct((NC, n_pad, d), jnp.float32),
            jax.ShapeDtypeStruct((NW, n_pad), jnp.float32),
        ],
        scratch_types=[
            pltpu.VMEM(((nch + 1) * CHUNK,), jnp.int32),
            pltpu.VMEM((2, CHUNK), jnp.int32),
            pltpu.VMEM((2, CHUNK), jnp.int32),
            pltpu.VMEM((2, CHUNK, d), jnp.float32),
            pltpu.VMEM((n_pad,), jnp.float32),
            pltpu.VMEM_SHARED((n_pad, d), jnp.float32),
            pltpu.SemaphoreType.DMA,
        ],
    )
    def k(data_h, pk_h, acc_o, cnt_o,
          pk_v, sidx_v, didx_v, rows_v, hist_v, acc_s, sem):
        c = lax.axis_index("c")
        s = lax.axis_index("s")
        wid = c * NS + s
        row0 = s * stripe

        zeros16 = jnp.zeros((16,), jnp.float32)

        def zhist(i, carry):
            hist_v[pl.ds(i * 16, 16)] = zeros16
            return carry

        lax.fori_loop(0, n_pad // 16, zhist, 0)

        def zrows(i, carry):
            for j in range(d // 16):
                rows_v[0, i, pl.ds(j * 16, 16)] = zeros16
            return carry

        lax.fori_loop(0, CHUNK, zrows, 0)

        nblk = n_pad // CHUNK

        def zacc(r, carry):
            bi = s + r * NS

            @pl.when(bi < nblk)
            def _():
                pltpu.sync_copy(rows_v.at[0],
                                acc_s.at[pl.ds(bi * CHUNK, CHUNK)])

            return carry

        lax.fori_loop(0, -(-nblk // NS), zacc, 0)
        pltpu.sync_copy(pk_h.at[wid], pk_v.at[pl.ds(0, nch * CHUNK)])
        dummy = jnp.full((16,), n_pad - 8, jnp.int32)
        for j in range(CHUNK // 16):
            pk_v[pl.ds(nch * CHUNK + j * 16, 16)] = dummy
        plsc.subcore_barrier()

        mask = jnp.int32((1 << PACK) - 1)
        ones16 = jnp.ones((16,), jnp.float32)

        def unpack(ci, sl):
            base = ci * CHUNK
            for j in range(CHUNK // 16):
                p = pk_v[pl.ds(base + j * 16, 16)]
                dstv = lax.bitwise_and(p, mask)
                sidx_v[sl, pl.ds(j * 16, 16)] = lax.shift_right_logical(p, PACK)
                didx_v[sl, pl.ds(j * 16, 16)] = dstv
                plsc.addupdate_scatter(hist_v, [dstv], ones16)

        def pair(kk, carry):
            c0 = 2 * kk
            unpack(c0, 0)
            d0 = pltpu.async_copy(data_h.at[sidx_v.at[0]], rows_v.at[0], sem)
            unpack(c0 + 1, 1)
            d1 = pltpu.async_copy(data_h.at[sidx_v.at[1]], rows_v.at[1], sem)
            d0.wait()
            pltpu.sync_copy(rows_v.at[0], acc_s.at[didx_v.at[0]], add=True)
            d1.wait()
            pltpu.sync_copy(rows_v.at[1], acc_s.at[didx_v.at[1]], add=True)
            return carry

        npair = lax.select(c == 0, jnp.int32(nch0 // 2), jnp.int32(nch1 // 2))
        lax.fori_loop(0, npair, pair, 0)
        plsc.subcore_barrier()

        pltpu.sync_copy(acc_s.at[pl.ds(row0, stripe)],
                        acc_o.at[c, pl.ds(row0, stripe)])
        pltpu.sync_copy(hist_v, cnt_o.at[wid])

    return k(data, packed)


def _tc_combine(data, merge, acc, cnt, w_lin, w_tr, b_tr):
    n, d = data.shape
    blk = 2048
    grid = -(-n // blk)
    bt2 = b_tr.reshape(1, d)
    onesw = jnp.ones((NW, 1), jnp.float32)

    def body(d_r, m_r, a_r, c_r, wl_r, wt_r, bt_r, ow_r, o_r):
        a = a_r[0, :, :] + a_r[1, :, :]
        deg = lax.dot_general(c_r[...], ow_r[...], (((0,), (0,)), ((), ())),
                              preferred_element_type=jnp.float32)
        factor = 1.0 / jnp.maximum(deg, 1.0)
        ind = (deg > 0.0).astype(jnp.float32)
        z = d_r[...] * ind - a * factor
        r = lax.dot_general(z, wl_r[...], (((1,), (1,)), ((), ())),
                            preferred_element_type=jnp.float32)
        r = r + lax.dot_general(m_r[...], wt_r[...], (((1,), (1,)), ((), ())),
                                preferred_element_type=jnp.float32)
        o_r[...] = jnp.maximum(r + bt_r[...], 0.0)

    return pl.pallas_call(
        body,
        grid=(grid,),
        in_specs=[
            pl.BlockSpec((blk, d), lambda i: (i, 0)),
            pl.BlockSpec((blk, d), lambda i: (i, 0)),
            pl.BlockSpec((NC, blk, d), lambda i: (0, i, 0)),
            pl.BlockSpec((NW, blk), lambda i: (0, i)),
            pl.BlockSpec((d, d), lambda i: (0, 0)),
            pl.BlockSpec((d, d), lambda i: (0, 0)),
            pl.BlockSpec((1, d), lambda i: (0, 0)),
            pl.BlockSpec((NW, 1), lambda i: (0, 0)),
        ],
        out_specs=pl.BlockSpec((blk, d), lambda i: (i, 0)),
        out_shape=jax.ShapeDtypeStruct((n, d), jnp.float32),
    )(data, merge, acc, cnt, w_lin, w_tr, bt2, onesw)


def kernel(data, merge, structure, W_lin, b_lin, W_tr, b_tr):
    n, d = data.shape
    e = structure.shape[1]

    FRAC0 = 2.0 / 3.0
    total_chunks = -(-e // CHUNK)
    nch0 = 2 * (-(-int(total_chunks * FRAC0) // (NS * 2)))
    rem = max(total_chunks - NS * nch0, 0)
    nch1 = 2 * (-(-rem // (NS * 2)))
    e0 = NS * nch0 * CHUNK
    e1 = NS * nch1 * CHUNK
    pad = e0 + e1 - e
    n_pad = NS * 8 * (-(-(n + 1) // (NS * 8)))

    src = structure[0]
    dst = structure[1]
    pad_dst = n + jnp.arange(pad, dtype=jnp.int32) % (n_pad - n)
    flat = jnp.concatenate([lax.shift_left(src, PACK) | dst, pad_dst])
    rows0 = flat[:e0].reshape(NS, nch0 * CHUNK)
    rows1 = jnp.pad(flat[e0:].reshape(NS, nch1 * CHUNK),
                    ((0, 0), (0, (nch0 - nch1) * CHUNK)))
    packed = jnp.concatenate([rows0, rows1])

    acc, cnt = _sc_segment_sum(data, packed, n_pad, nch0, nch1)
    return _tc_combine(data, merge, acc, cnt, W_lin, W_tr, b_tr)


# speedup vs baseline: 19.7437x; 1.0513x over previous
import functools

import jax
import jax.numpy as jnp
from jax import lax
from jax.experimental import pallas as pl
from jax.experimental.pallas import tpu as pltpu
from jax.experimental.pallas import tpu_sc as plsc

NC = 2
NS = 16
NW = NC * NS
CHUNK = 64
PACK = 14


def _sc_segment_sum(data, packed, n_pad, nch0, nch1):
    nch = nch0
    d = data.shape[1]
    stripe = n_pad // NS
    mesh = plsc.VectorSubcoreMesh(core_axis_name="c", subcore_axis_name="s")

    @functools.partial(
        pl.kernel,
        mesh=mesh,
        compiler_params=pltpu.CompilerParams(needs_layout_passes=False),
        out_type=[
            jax.ShapeDtypeStruct((NC, n_pad, d), jnp.float32),
            jax.ShapeDtypeStruct((NW, n_pad), jnp.float32),
        ],
        scratch_types=[
            pltpu.VMEM(((nch + 1) * CHUNK,), jnp.int32),
            pltpu.VMEM((2, CHUNK), jnp.int32),
            pltpu.VMEM((2, CHUNK), jnp.int32),
            pltpu.VMEM((2, CHUNK, d), jnp.float32),
            pltpu.VMEM((n_pad,), jnp.float32),
            pltpu.VMEM_SHARED((n_pad, d), jnp.float32),
            pltpu.SemaphoreType.DMA,
        ],
    )
    def k(data_h, pk_h, acc_o, cnt_o,
          pk_v, sidx_v, didx_v, rows_v, hist_v, acc_s, sem):
        c = lax.axis_index("c")
        s = lax.axis_index("s")
        wid = c * NS + s
        row0 = s * stripe

        zeros16 = jnp.zeros((16,), jnp.float32)

        def zhist(i, carry):
            hist_v[pl.ds(i * 16, 16)] = zeros16
            return carry

        lax.fori_loop(0, n_pad // 16, zhist, 0)

        def zrows(i, carry):
            for j in range(d // 16):
                rows_v[0, i, pl.ds(j * 16, 16)] = zeros16
            return carry

        lax.fori_loop(0, CHUNK, zrows, 0)

        nblk = n_pad // CHUNK

        def zacc(r, carry):
            bi = s + r * NS

            @pl.when(bi < nblk)
            def _():
                pltpu.sync_copy(rows_v.at[0],
                                acc_s.at[pl.ds(bi * CHUNK, CHUNK)])

            return carry

        lax.fori_loop(0, -(-nblk // NS), zacc, 0)
        pltpu.sync_copy(pk_h.at[wid], pk_v.at[pl.ds(0, nch * CHUNK)])
        dummy = jnp.full((16,), n_pad - 8, jnp.int32)
        for j in range(CHUNK // 16):
            pk_v[pl.ds(nch * CHUNK + j * 16, 16)] = dummy
        plsc.subcore_barrier()

        mask = jnp.int32((1 << PACK) - 1)
        ones16 = jnp.ones((16,), jnp.float32)

        def unpack(ci, sl):
            base = ci * CHUNK
            for j in range(CHUNK // 16):
                p = pk_v[pl.ds(base + j * 16, 16)]
                dstv = lax.bitwise_and(p, mask)
                sidx_v[sl, pl.ds(j * 16, 16)] = lax.shift_right_logical(p, PACK)
                didx_v[sl, pl.ds(j * 16, 16)] = dstv
                plsc.addupdate_scatter(hist_v, [dstv], ones16)

        def pair(kk, carry):
            c0 = 2 * kk
            unpack(c0, 0)
            d0 = pltpu.async_copy(data_h.at[sidx_v.at[0]], rows_v.at[0], sem)
            unpack(c0 + 1, 1)
            d1 = pltpu.async_copy(data_h.at[sidx_v.at[1]], rows_v.at[1], sem)
            d0.wait()
            pltpu.sync_copy(rows_v.at[0], acc_s.at[didx_v.at[0]], add=True)
            d1.wait()
            pltpu.sync_copy(rows_v.at[1], acc_s.at[didx_v.at[1]], add=True)
            return carry

        npair = lax.select(c == 0, jnp.int32(nch0 // 2), jnp.int32(nch1 // 2))
        lax.fori_loop(0, npair, pair, 0)
        plsc.subcore_barrier()

        pltpu.sync_copy(acc_s.at[pl.ds(row0, stripe)],
                        acc_o.at[c, pl.ds(row0, stripe)])
        pltpu.sync_copy(hist_v, cnt_o.at[wid])

    return k(data, packed)


def _tc_combine(data, merge, acc, cnt, w_lin, w_tr, b_tr):
    n, d = data.shape
    blk = 2048
    grid = -(-n // blk)
    bt2 = b_tr.reshape(1, d)
    onesw = jnp.ones((NW, 1), jnp.float32)

    def body(d_r, m_r, a_r, c_r, wl_r, wt_r, bt_r, ow_r, o_r):
        a = a_r[0, :, :] + a_r[1, :, :]
        deg = lax.dot_general(c_r[...], ow_r[...], (((0,), (0,)), ((), ())),
                              preferred_element_type=jnp.float32)
        factor = 1.0 / jnp.maximum(deg, 1.0)
        ind = (deg > 0.0).astype(jnp.float32)
        z = d_r[...] * ind - a * factor
        r = lax.dot_general(z, wl_r[...], (((1,), (1,)), ((), ())),
                            preferred_element_type=jnp.float32)
        r = r + lax.dot_general(m_r[...], wt_r[...], (((1,), (1,)), ((), ())),
                                preferred_element_type=jnp.float32)
        o_r[...] = jnp.maximum(r + bt_r[...], 0.0)

    return pl.pallas_call(
        body,
        grid=(grid,),
        in_specs=[
            pl.BlockSpec((blk, d), lambda i: (i, 0)),
            pl.BlockSpec((blk, d), lambda i: (i, 0)),
            pl.BlockSpec((NC, blk, d), lambda i: (0, i, 0)),
            pl.BlockSpec((NW, blk), lambda i: (0, i)),
            pl.BlockSpec((d, d), lambda i: (0, 0)),
            pl.BlockSpec((d, d), lambda i: (0, 0)),
            pl.BlockSpec((1, d), lambda i: (0, 0)),
            pl.BlockSpec((NW, 1), lambda i: (0, 0)),
        ],
        out_specs=pl.BlockSpec((blk, d), lambda i: (i, 0)),
        out_shape=jax.ShapeDtypeStruct((n, d), jnp.float32),
    )(data, merge, acc, cnt, w_lin, w_tr, bt2, onesw)


def kernel(data, merge, structure, W_lin, b_lin, W_tr, b_tr):
    n, d = data.shape
    e = structure.shape[1]

    FRAC0 = 0.623
    total_chunks = -(-e // CHUNK)
    nch0 = 2 * (-(-int(total_chunks * FRAC0) // (NS * 2)))
    rem = max(total_chunks - NS * nch0, 0)
    nch1 = 2 * (-(-rem // (NS * 2)))
    e0 = NS * nch0 * CHUNK
    e1 = NS * nch1 * CHUNK
    pad = e0 + e1 - e
    n_pad = NS * 8 * (-(-(n + 1) // (NS * 8)))

    src = structure[0]
    dst = structure[1]
    pad_dst = n + jnp.arange(pad, dtype=jnp.int32) % (n_pad - n)
    flat = jnp.concatenate([lax.shift_left(src, PACK) | dst, pad_dst])
    rows0 = flat[:e0].reshape(NS, nch0 * CHUNK)
    rows1 = jnp.pad(flat[e0:].reshape(NS, nch1 * CHUNK),
                    ((0, 0), (0, (nch0 - nch1) * CHUNK)))
    packed = jnp.concatenate([rows0, rows1])

    acc, cnt = _sc_segment_sum(data, packed, n_pad, nch0, nch1)
    return _tc_combine(data, merge, acc, cnt, W_lin, W_tr, b_tr)


# speedup vs baseline: 19.7702x; 1.0013x over previous
import functools

import jax
import jax.numpy as jnp
from jax import lax
from jax.experimental import pallas as pl
from jax.experimental.pallas import tpu as pltpu
from jax.experimental.pallas import tpu_sc as plsc

NC = 2
NS = 16
NW = NC * NS
CHUNK = 64
PACK = 14


def _sc_segment_sum(data, packed, n_pad, nch0, nch1):
    nch = nch0
    d = data.shape[1]
    stripe = n_pad // NS
    mesh = plsc.VectorSubcoreMesh(core_axis_name="c", subcore_axis_name="s")

    @functools.partial(
        pl.kernel,
        mesh=mesh,
        compiler_params=pltpu.CompilerParams(needs_layout_passes=False),
        out_type=[
            jax.ShapeDtypeStruct((NC, n_pad, d), jnp.float32),
            jax.ShapeDtypeStruct((NW, n_pad), jnp.float32),
        ],
        scratch_types=[
            pltpu.VMEM(((nch + 1) * CHUNK,), jnp.int32),
            pltpu.VMEM((2, CHUNK), jnp.int32),
            pltpu.VMEM((2, CHUNK), jnp.int32),
            pltpu.VMEM((2, CHUNK, d), jnp.float32),
            pltpu.VMEM((n_pad,), jnp.float32),
            pltpu.VMEM_SHARED((n_pad, d), jnp.float32),
            pltpu.SemaphoreType.DMA,
        ],
    )
    def k(data_h, pk_h, acc_o, cnt_o,
          pk_v, sidx_v, didx_v, rows_v, hist_v, acc_s, sem):
        c = lax.axis_index("c")
        s = lax.axis_index("s")
        wid = c * NS + s
        row0 = s * stripe

        zeros16 = jnp.zeros((16,), jnp.float32)

        def zhist(i, carry):
            hist_v[pl.ds(i * 16, 16)] = zeros16
            return carry

        lax.fori_loop(0, n_pad // 16, zhist, 0)

        def zrows(i, carry):
            for j in range(d // 16):
                rows_v[0, i, pl.ds(j * 16, 16)] = zeros16
            return carry

        lax.fori_loop(0, CHUNK, zrows, 0)

        nblk = n_pad // CHUNK

        def zacc(r, carry):
            bi = s + r * NS

            @pl.when(bi < nblk)
            def _():
                pltpu.sync_copy(rows_v.at[0],
                                acc_s.at[pl.ds(bi * CHUNK, CHUNK)])

            return carry

        lax.fori_loop(0, -(-nblk // NS), zacc, 0)
        pltpu.sync_copy(pk_h.at[wid], pk_v.at[pl.ds(0, nch * CHUNK)])
        dummy = jnp.full((16,), n_pad - 8, jnp.int32)
        for j in range(CHUNK // 16):
            pk_v[pl.ds(nch * CHUNK + j * 16, 16)] = dummy
        plsc.subcore_barrier()

        mask = jnp.int32((1 << PACK) - 1)
        ones16 = jnp.ones((16,), jnp.float32)

        def unpack(ci, sl):
            base = ci * CHUNK
            for j in range(CHUNK // 16):
                p = pk_v[pl.ds(base + j * 16, 16)]
                dstv = lax.bitwise_and(p, mask)
                sidx_v[sl, pl.ds(j * 16, 16)] = lax.shift_right_logical(p, PACK)
                didx_v[sl, pl.ds(j * 16, 16)] = dstv
                plsc.addupdate_scatter(hist_v, [dstv], ones16)

        def pair(kk, carry):
            c0 = 2 * kk
            descs = []
            for b in range(2):
                unpack(c0 + b, b)
                descs.append(pltpu.async_copy(data_h.at[sidx_v.at[b]],
                                              rows_v.at[b], sem))
            for b in range(2):
                descs[b].wait()
                pltpu.sync_copy(rows_v.at[b], acc_s.at[didx_v.at[b]], add=True)
            return carry

        npair = lax.select(c == 0, jnp.int32(nch0 // 2), jnp.int32(nch1 // 2))
        lax.fori_loop(0, npair, pair, 0)
        plsc.subcore_barrier()

        pltpu.sync_copy(acc_s.at[pl.ds(row0, stripe)],
                        acc_o.at[c, pl.ds(row0, stripe)])
        pltpu.sync_copy(hist_v, cnt_o.at[wid])

    return k(data, packed)


def _tc_combine(data, merge, acc, cnt, w_lin, w_tr, b_tr):
    n, d = data.shape
    blk = 2048
    grid = -(-n // blk)
    bt2 = b_tr.reshape(1, d)
    onesw = jnp.ones((NW, 1), jnp.float32)

    def body(d_r, m_r, a_r, c_r, wl_r, wt_r, bt_r, ow_r, o_r):
        a = a_r[0, :, :] + a_r[1, :, :]
        deg = lax.dot_general(c_r[...], ow_r[...], (((0,), (0,)), ((), ())),
                              preferred_element_type=jnp.float32)
        factor = 1.0 / jnp.maximum(deg, 1.0)
        ind = (deg > 0.0).astype(jnp.float32)
        z = d_r[...] * ind - a * factor
        r = lax.dot_general(z, wl_r[...], (((1,), (1,)), ((), ())),
                            preferred_element_type=jnp.float32)
        r = r + lax.dot_general(m_r[...], wt_r[...], (((1,), (1,)), ((), ())),
                                preferred_element_type=jnp.float32)
        o_r[...] = jnp.maximum(r + bt_r[...], 0.0)

    return pl.pallas_call(
        body,
        grid=(grid,),
        in_specs=[
            pl.BlockSpec((blk, d), lambda i: (i, 0)),
            pl.BlockSpec((blk, d), lambda i: (i, 0)),
            pl.BlockSpec((NC, blk, d), lambda i: (0, i, 0)),
            pl.BlockSpec((NW, blk), lambda i: (0, i)),
            pl.BlockSpec((d, d), lambda i: (0, 0)),
            pl.BlockSpec((d, d), lambda i: (0, 0)),
            pl.BlockSpec((1, d), lambda i: (0, 0)),
            pl.BlockSpec((NW, 1), lambda i: (0, 0)),
        ],
        out_specs=pl.BlockSpec((blk, d), lambda i: (i, 0)),
        out_shape=jax.ShapeDtypeStruct((n, d), jnp.float32),
    )(data, merge, acc, cnt, w_lin, w_tr, bt2, onesw)


def kernel(data, merge, structure, W_lin, b_lin, W_tr, b_tr):
    n, d = data.shape
    e = structure.shape[1]

    FRAC0 = 0.623
    total_chunks = -(-e // CHUNK)
    nch0 = 2 * (-(-int(total_chunks * FRAC0) // (NS * 2)))
    rem = max(total_chunks - NS * nch0, 0)
    nch1 = 2 * (-(-rem // (NS * 2)))
    e0 = NS * nch0 * CHUNK
    e1 = NS * nch1 * CHUNK
    pad = e0 + e1 - e
    n_pad = NS * 8 * (-(-(n + 1) // (NS * 8)))

    src = structure[0]
    dst = structure[1]
    pad_dst = n + jnp.arange(pad, dtype=jnp.int32) % (n_pad - n)
    flat = jnp.concatenate([lax.shift_left(src, PACK) | dst, pad_dst])
    rows0 = flat[:e0].reshape(NS, nch0 * CHUNK)
    rows1 = jnp.pad(flat[e0:].reshape(NS, nch1 * CHUNK),
                    ((0, 0), (0, (nch0 - nch1) * CHUNK)))
    packed = jnp.concatenate([rows0, rows1])

    acc, cnt = _sc_segment_sum(data, packed, n_pad, nch0, nch1)
    return _tc_combine(data, merge, acc, cnt, W_lin, W_tr, b_tr)
